# baseline (speedup 1.0000x reference)
import functools

import jax
import jax.numpy as jnp
from jax import lax
from jax.experimental import pallas as pl
from jax.experimental.pallas import tpu as pltpu
from jax.experimental.pallas import tpu_sc as plsc

N = 50000
E = 800000
NFEAT = 128
NHID = 128
NCLASS = 64
K = 8

NC = 2
NS = 16
LANES = 16
FH = 32

ROWS_PER_TILE = 1600
NPAD = NC * NS * ROWS_PER_TILE
HALF = NS * ROWS_PER_TILE

EC = 128
CROWS = 4
CE = CROWS * EC
NCH = 100
NQUAD = NCH // 4
EDGES_PER_TILE = NCH * CE
EPAD = NS * EDGES_PER_TILE
EROWS = EPAD // EC
NBUF = 4


def _mlp_body(x_ref, w1_ref, b1_ref, w2_ref, b2_ref, o0_ref, o1_ref):
    h = jnp.dot(x_ref[...], w1_ref[...], preferred_element_type=jnp.float32)
    h = jnp.maximum(h + b1_ref[...], 0.0)
    o = jnp.dot(h, w2_ref[...], preferred_element_type=jnp.float32)
    o = o + b2_ref[...]
    o0_ref[...] = o[:, :FH]
    o1_ref[...] = o[:, FH:]


_MLP_BM = 2048
_MLP_GRID = NPAD // _MLP_BM


def _mlp(xp, W1, b1, W2, b2):
    return pl.pallas_call(
        _mlp_body,
        grid=(_MLP_GRID,),
        in_specs=[
            pl.BlockSpec((_MLP_BM, NFEAT), lambda i: (i, 0)),
            pl.BlockSpec((NFEAT, NHID), lambda i: (0, 0)),
            pl.BlockSpec((1, NHID), lambda i: (0, 0)),
            pl.BlockSpec((NHID, NCLASS), lambda i: (0, 0)),
            pl.BlockSpec((1, NCLASS), lambda i: (0, 0)),
        ],
        out_specs=[
            pl.BlockSpec((_MLP_BM, FH), lambda i: (i, 0)),
            pl.BlockSpec((_MLP_BM, FH), lambda i: (i, 0)),
        ],
        out_shape=[
            jax.ShapeDtypeStruct((NPAD, FH), jnp.float32),
            jax.ShapeDtypeStruct((NPAD, FH), jnp.float32),
        ],
    )(xp, W1, b1.reshape(1, NHID), W2, b2.reshape(1, NCLASS))


def _hop_body(src_hbm, dst_hbm, val_hbm, in0_hbm, in1_hbm, zeros_hbm,
              out0_hbm, out1_hbm, acc_sh, srcb, dstb, valb, rowsb, *sems):
    esem = sems[0:NBUF]
    gsem = sems[NBUF:2 * NBUF]
    ssem = sems[2 * NBUF:3 * NBUF]
    core = lax.axis_index("c")
    sid = lax.axis_index("s")
    base = core * HALF
    row0 = sid * (NCH * CROWS)

    edge_bufs = ((src_hbm, srcb), (dst_hbm, dstb), (val_hbm, valb))

    def fire_edges(x, r):
        for h, b in edge_bufs:
            pltpu.async_copy(h.at[pl.ds(r, CROWS)], b.at[x], esem[x])

    def drain_edges(x):
        for h, b in edge_bufs:
            pltpu.make_async_copy(h.at[pl.ds(row0, CROWS)], b.at[x], esem[x]).wait()

    def mask_buf(x):
        for j in range(CROWS):
            for g in range(EC // LANES):
                sl = pl.ds(g * LANES, LANES)
                rel = dstb[x, j, sl] - base
                inr = (rel >= 0) & (rel < HALF)
                dstb[x, j, sl] = jnp.where(inr, rel, 0)
                valb[x, j, sl] = jnp.where(inr, valb[x, j, sl], 0.0)

    def mult_buf(x):
        for j in range(CROWS):
            def mbody(g, _, j=j):
                g16 = pl.multiple_of(g * LANES, LANES)
                v16 = valb[x, j, pl.ds(g16, LANES)]
                for l in range(LANES):
                    vb = jnp.full((LANES,), v16[l], jnp.float32)
                    e = j * EC + g16 + l
                    for c in range(FH // LANES):
                        sl = pl.ds(c * LANES, LANES)
                        rowsb[x, e, sl] = rowsb[x, e, sl] * vb
                return 0
            lax.fori_loop(0, EC // LANES, mbody, 0)

    def run_pass(in_hbm, out_hbm):
        def fire_gather(x):
            for j in range(CROWS):
                pltpu.async_copy(in_hbm.at[srcb.at[x, j]],
                                 rowsb.at[x, pl.ds(j * EC, EC)], gsem[x])

        def drain_gather(x):
            for j in range(CROWS):
                pltpu.make_async_copy(in_hbm.at[srcb.at[x, j]],
                                      rowsb.at[x, pl.ds(j * EC, EC)],
                                      gsem[x]).wait()

        def fire_scatter(x):
            for j in range(CROWS):
                pltpu.async_copy(rowsb.at[x, pl.ds(j * EC, EC)],
                                 acc_sh.at[dstb.at[x, j]], ssem[x], add=True)

        def drain_scatter(x):
            for j in range(CROWS):
                pltpu.make_async_copy(rowsb.at[x, pl.ds(j * EC, EC)],
                                      acc_sh.at[dstb.at[x, j]], ssem[x]).wait()

        pltpu.sync_copy(zeros_hbm,
                        acc_sh.at[pl.ds(sid * ROWS_PER_TILE, ROWS_PER_TILE)])
        plsc.subcore_barrier()

        fire_edges(0, row0)
        fire_edges(1, row0 + CROWS)
        fire_edges(2, row0 + 2 * CROWS)
        drain_edges(0)
        mask_buf(0)
        fire_gather(0)
        drain_edges(1)
        mask_buf(1)
        fire_gather(1)

        def quad(i, _):
            for s in range(NBUF):
                k = i * NBUF + s
                n2 = (s + 2) % NBUF
                n3 = (s + 3) % NBUF

                drain_gather(s)
                mult_buf(s)
                fire_scatter(s)

                @pl.when(k < NCH - 2)
                def _():
                    drain_edges(n2)
                    mask_buf(n2)
                    fire_gather(n2)

                @pl.when(k >= 1)
                def _():
                    drain_scatter(n3)

                @pl.when(k < NCH - 3)
                def _():
                    fire_edges(n3, row0 + (k + 3) * CROWS)
            return 0

        lax.fori_loop(0, NQUAD, quad, 0)
        drain_scatter((NCH - 1) % NBUF)
        plsc.subcore_barrier()

        pltpu.sync_copy(
            acc_sh.at[pl.ds(sid * ROWS_PER_TILE, ROWS_PER_TILE)],
            out_hbm.at[pl.ds(base + sid * ROWS_PER_TILE, ROWS_PER_TILE)])

    run_pass(in0_hbm, out0_hbm)
    run_pass(in1_hbm, out1_hbm)


_hop = functools.partial(
    pl.kernel,
    out_type=(
        jax.ShapeDtypeStruct((NPAD, FH), jnp.float32),
        jax.ShapeDtypeStruct((NPAD, FH), jnp.float32),
    ),
    mesh=plsc.VectorSubcoreMesh(core_axis_name="c", subcore_axis_name="s"),
    compiler_params=pltpu.CompilerParams(use_tc_tiling_on_sc=False),
    scratch_types=[
        pltpu.VMEM_SHARED((HALF, FH), jnp.float32),
        pltpu.VMEM((NBUF, CROWS, EC), jnp.int32),
        pltpu.VMEM((NBUF, CROWS, EC), jnp.int32),
        pltpu.VMEM((NBUF, CROWS, EC), jnp.float32),
        pltpu.VMEM((NBUF, CE, FH), jnp.float32),
    ] + [pltpu.SemaphoreType.DMA] * (3 * NBUF),
)(_hop_body)


def kernel(x, adj_values, W1, b1, W2, b2, adj_indices):
    dst = adj_indices[0]
    src = adj_indices[1]
    epad = EPAD - E
    srcp = jnp.concatenate([src, jnp.zeros((epad,), jnp.int32)]).reshape(EROWS, EC)
    dstp = jnp.concatenate([dst, jnp.zeros((epad,), jnp.int32)]).reshape(EROWS, EC)
    valp = jnp.concatenate(
        [adj_values, jnp.zeros((epad,), jnp.float32)]).reshape(EROWS, EC)
    xp = jnp.pad(x, ((0, NPAD - N), (0, 0)))
    zeros_blob = jnp.zeros((ROWS_PER_TILE, FH), jnp.float32)

    h0, h1 = _mlp(xp, W1, b1, W2, b2)
    for _ in range(K):
        h0, h1 = _hop(srcp, dstp, valp, h0, h1, zeros_blob)
    return jnp.concatenate([h0[:N], h1[:N]], axis=1)


# speedup vs baseline: 1.0561x; 1.0561x over previous
import functools

import jax
import jax.numpy as jnp
from jax import lax
from jax.experimental import pallas as pl
from jax.experimental.pallas import tpu as pltpu
from jax.experimental.pallas import tpu_sc as plsc

N = 50000
E = 800000
NFEAT = 128
NHID = 128
NCLASS = 64
K = 8

NC = 2
NS = 16
LANES = 16

ROWS_PER_TILE = 1600
NPAD = NC * NS * ROWS_PER_TILE
HALF = NS * ROWS_PER_TILE

EC = 128
CAP_ROWS = 6400
CAPE = CAP_ROWS * EC
NBUF = 3


def _mlp_body(x_ref, w1_ref, b1_ref, w2_ref, b2_ref, o_ref):
    h = jnp.dot(x_ref[...], w1_ref[...], preferred_element_type=jnp.float32)
    h = jnp.maximum(h + b1_ref[...], 0.0)
    o = jnp.dot(h, w2_ref[...], preferred_element_type=jnp.float32)
    o_ref[...] = o + b2_ref[...]


_MLP_BM = 2048
_MLP_GRID = NPAD // _MLP_BM


def _mlp(xp, W1, b1, W2, b2):
    return pl.pallas_call(
        _mlp_body,
        grid=(_MLP_GRID,),
        in_specs=[
            pl.BlockSpec((_MLP_BM, NFEAT), lambda i: (i, 0)),
            pl.BlockSpec((NFEAT, NHID), lambda i: (0, 0)),
            pl.BlockSpec((1, NHID), lambda i: (0, 0)),
            pl.BlockSpec((NHID, NCLASS), lambda i: (0, 0)),
            pl.BlockSpec((1, NCLASS), lambda i: (0, 0)),
        ],
        out_specs=pl.BlockSpec((_MLP_BM, NCLASS), lambda i: (i, 0)),
        out_shape=jax.ShapeDtypeStruct((NPAD, NCLASS), jnp.float32),
    )(xp, W1, b1.reshape(1, NHID), W2, b2.reshape(1, NCLASS))


def _hop_body(edges_hbm, meta_hbm, in_hbm, zeros_hbm, out_hbm,
              acc_sh, ebuf, rowsb, meta_v, *sems):
    esem = sems[0:NBUF]
    gsem = sems[NBUF:2 * NBUF]
    ssem = sems[2 * NBUF:3 * NBUF]
    core = lax.axis_index("c")
    sid = lax.axis_index("s")
    base = core * HALF
    row_base = core * CAP_ROWS

    pltpu.sync_copy(meta_hbm, meta_v)
    mvec = meta_v[...]
    nch = jnp.where(core == 0, mvec[0], mvec[1])
    m = (nch - sid + 15) // 16
    ntrip = ((m + 2) * 21846) >> 16

    def erow(t):
        return row_base + sid + t * 16

    def fire_edges(x, t):
        pltpu.async_copy(edges_hbm.at[erow(t)], ebuf.at[x], esem[x])

    def drain_edges(x):
        pltpu.make_async_copy(edges_hbm.at[row_base], ebuf.at[x], esem[x]).wait()

    def fire_gather(x):
        pltpu.async_copy(in_hbm.at[ebuf.at[x, 0]], rowsb.at[x], gsem[x])

    def drain_gather(x):
        pltpu.make_async_copy(in_hbm.at[ebuf.at[x, 0]], rowsb.at[x],
                              gsem[x]).wait()

    def fire_scatter(x):
        pltpu.async_copy(rowsb.at[x], acc_sh.at[ebuf.at[x, 1]], ssem[x],
                         add=True)

    def drain_scatter(x):
        pltpu.make_async_copy(rowsb.at[x], acc_sh.at[ebuf.at[x, 1]],
                              ssem[x]).wait()

    def mult_buf(x):
        def mbody(g, _):
            g16 = pl.multiple_of(g * LANES, LANES)
            v16 = ebuf[x, 2, pl.ds(g16, LANES)].astype(jnp.float32) * jnp.float32(2.0 ** -26)
            for l in range(LANES):
                vb = jnp.full((LANES,), v16[l], jnp.float32)
                e = g16 + l
                for c in range(NCLASS // LANES):
                    sl = pl.ds(c * LANES, LANES)
                    rowsb[x, e, sl] = rowsb[x, e, sl] * vb
            return 0
        lax.fori_loop(0, EC // LANES, mbody, 0)

    pltpu.sync_copy(zeros_hbm,
                    acc_sh.at[pl.ds(sid * ROWS_PER_TILE, ROWS_PER_TILE)])
    plsc.subcore_barrier()

    fire_edges(0, 0)
    fire_edges(1, 1)
    drain_edges(0)
    fire_gather(0)

    def triple(i, _):
        for s in range(NBUF):
            t = i * NBUF + s
            nxt = (s + 1) % NBUF
            prv = (s + 2) % NBUF

            @pl.when(t < m - 1)
            def _():
                drain_edges(nxt)
                fire_gather(nxt)

            @pl.when(t < m)
            def _():
                drain_gather(s)
                mult_buf(s)
                fire_scatter(s)

            @pl.when((t >= 1) & (t < m))
            def _():
                drain_scatter(prv)

            @pl.when(t < m - 2)
            def _():
                fire_edges(prv, t + 2)
        return 0

    lax.fori_loop(0, ntrip, triple, 0)
    last = (m - 1) - NBUF * (((m - 1) * 21846) >> 16)
    for s in range(NBUF):
        @pl.when(last == s)
        def _():
            drain_scatter(s)

    @pl.when(m == 1)
    def _():
        drain_edges(1)

    plsc.subcore_barrier()

    pltpu.sync_copy(
        acc_sh.at[pl.ds(sid * ROWS_PER_TILE, ROWS_PER_TILE)],
        out_hbm.at[pl.ds(base + sid * ROWS_PER_TILE, ROWS_PER_TILE)])


_hop = functools.partial(
    pl.kernel,
    out_type=jax.ShapeDtypeStruct((NPAD, NCLASS), jnp.float32),
    mesh=plsc.VectorSubcoreMesh(core_axis_name="c", subcore_axis_name="s"),
    compiler_params=pltpu.CompilerParams(use_tc_tiling_on_sc=False),
    scratch_types=[
        pltpu.VMEM_SHARED((HALF, NCLASS), jnp.float32),
        pltpu.VMEM((NBUF, 3, EC), jnp.int32),
        pltpu.VMEM((NBUF, EC, NCLASS), jnp.float32),
        pltpu.VMEM((16,), jnp.int32),
    ] + [pltpu.SemaphoreType.DMA] * (3 * NBUF),
)(_hop_body)


def kernel(x, adj_values, W1, b1, W2, b2, adj_indices):
    dst = adj_indices[0]
    src = adj_indices[1]

    side = (dst >= HALF).astype(jnp.int32)
    keep0 = 1 - side
    idx0 = jnp.cumsum(keep0) - keep0
    idx1 = jnp.cumsum(side) - side
    cnt0 = jnp.sum(keep0)
    pos = jnp.where(side == 1, CAPE + idx1, idx0)
    dst_rel = dst - side * HALF

    def scatter_part(vals, dtype):
        buf = jnp.zeros((2 * CAPE,), dtype)
        return buf.at[pos].set(vals, mode="promise_in_bounds",
                               unique_indices=True)

    srcq = scatter_part(src, jnp.int32).reshape(2 * CAP_ROWS, EC)
    dstq = scatter_part(dst_rel, jnp.int32).reshape(2 * CAP_ROWS, EC)
    val_fix = (adj_values * jnp.float32(2.0 ** 26)).astype(jnp.int32)
    valq = scatter_part(val_fix, jnp.int32).reshape(2 * CAP_ROWS, EC)
    edges = jnp.stack([srcq, dstq, valq], axis=1)

    cnt1 = E - cnt0
    nch0 = jnp.clip((cnt0 + EC - 1) // EC, 16, CAP_ROWS).astype(jnp.int32)
    nch1 = jnp.clip((cnt1 + EC - 1) // EC, 16, CAP_ROWS).astype(jnp.int32)
    meta = jnp.zeros((16,), jnp.int32).at[0].set(nch0).at[1].set(nch1)

    xp = jnp.pad(x, ((0, NPAD - N), (0, 0)))
    zeros_blob = jnp.zeros((ROWS_PER_TILE, NCLASS), jnp.float32)

    h = _mlp(xp, W1, b1, W2, b2)
    for _ in range(K):
        h = _hop(edges, meta, h, zeros_blob)
    return h[:N]


# speedup vs baseline: 3.1223x; 2.9565x over previous
import functools

import jax
import jax.numpy as jnp
from jax import lax
from jax.experimental import pallas as pl
from jax.experimental.pallas import tpu as pltpu
from jax.experimental.pallas import tpu_sc as plsc

N = 50000
E = 800000
NFEAT = 128
NHID = 128
NCLASS = 64
K = 8

NC = 2
NS = 16
LANES = 16
FH = 32

NPAD = 51200
ROWS_PER_TILE = NPAD // NS

EC = 128
CROWS = 2
CE = CROWS * EC
NCHUNKS = 198
NTRIP = NCHUNKS // 3
ROWS_PER_TILE_E = NCHUNKS * CROWS
EROWS = NS * ROWS_PER_TILE_E
EPAD = EROWS * EC
NBUF = 3
VSCALE = 2.0 ** 26


def _mlp_body(x_ref, w1_ref, b1_ref, w2_ref, b2_ref, o_ref):
    h = jnp.dot(x_ref[...], w1_ref[...], preferred_element_type=jnp.float32)
    h = jnp.maximum(h + b1_ref[...], 0.0)
    o = jnp.dot(h, w2_ref[...], preferred_element_type=jnp.float32)
    o = o + b2_ref[...]
    o_ref[0] = o[:, :FH]
    o_ref[1] = o[:, FH:]


_MLP_BM = 2048
_MLP_GRID = NPAD // _MLP_BM


def _mlp(xp, W1, b1, W2, b2):
    return pl.pallas_call(
        _mlp_body,
        grid=(_MLP_GRID,),
        in_specs=[
            pl.BlockSpec((_MLP_BM, NFEAT), lambda i: (i, 0)),
            pl.BlockSpec((NFEAT, NHID), lambda i: (0, 0)),
            pl.BlockSpec((1, NHID), lambda i: (0, 0)),
            pl.BlockSpec((NHID, NCLASS), lambda i: (0, 0)),
            pl.BlockSpec((1, NCLASS), lambda i: (0, 0)),
        ],
        out_specs=pl.BlockSpec((2, _MLP_BM, FH), lambda i: (0, i, 0)),
        out_shape=jax.ShapeDtypeStruct((2, NPAD, FH), jnp.float32),
    )(xp, W1, b1.reshape(1, NHID), W2, b2.reshape(1, NCLASS))


def _hop_body(edges_hbm, in0_hbm, in1_hbm, zeros_hbm, out_hbm,
              acc_sh, ebuf, rowsb, *sems):
    esem = sems[0:NBUF]
    gsem = sems[NBUF:2 * NBUF]
    ssem = sems[2 * NBUF:3 * NBUF]
    core = lax.axis_index("c")
    sid = lax.axis_index("s")
    erow0 = sid * ROWS_PER_TILE_E

    def fire_edges(x, t):
        pltpu.async_copy(edges_hbm.at[pl.ds(erow0 + t * CROWS, CROWS)],
                         ebuf.at[x], esem[x])

    def drain_edges(x):
        pltpu.make_async_copy(edges_hbm.at[pl.ds(erow0, CROWS)], ebuf.at[x],
                              esem[x]).wait()

    def mult_buf(x):
        for j in range(CROWS):
            def mbody(g, _, j=j):
                g16 = pl.multiple_of(g * LANES, LANES)
                v16 = ebuf[x, j, 2, pl.ds(g16, LANES)].astype(jnp.float32)
                v16 = v16 * jnp.float32(1.0 / VSCALE)
                for l in range(LANES):
                    vb = jnp.full((LANES,), v16[l], jnp.float32)
                    e = j * EC + g16 + l
                    for c in range(FH // LANES):
                        sl = pl.ds(c * LANES, LANES)
                        rowsb[x, e, sl] = rowsb[x, e, sl] * vb
                return 0
            lax.fori_loop(0, EC // LANES, mbody, 0)

    def run_pass(in_hbm):
        def fire_gather(x):
            for j in range(CROWS):
                pltpu.async_copy(in_hbm.at[ebuf.at[x, j, 0]],
                                 rowsb.at[x, pl.ds(j * EC, EC)], gsem[x])

        def drain_gather(x):
            for j in range(CROWS):
                pltpu.make_async_copy(in_hbm.at[ebuf.at[x, j, 0]],
                                      rowsb.at[x, pl.ds(j * EC, EC)],
                                      gsem[x]).wait()

        def fire_scatter(x):
            for j in range(CROWS):
                pltpu.async_copy(rowsb.at[x, pl.ds(j * EC, EC)],
                                 acc_sh.at[ebuf.at[x, j, 1]], ssem[x],
                                 add=True)

        def drain_scatter(x):
            for j in range(CROWS):
                pltpu.make_async_copy(rowsb.at[x, pl.ds(j * EC, EC)],
                                      acc_sh.at[ebuf.at[x, j, 1]],
                                      ssem[x]).wait()

        fire_edges(0, 0)
        fire_edges(1, 1)
        drain_edges(0)
        fire_gather(0)

        def triple(i, _):
            for s in range(NBUF):
                t = i * NBUF + s
                nxt = (s + 1) % NBUF
                prv = (s + 2) % NBUF

                @pl.when(t < NCHUNKS - 1)
                def _():
                    drain_edges(nxt)
                    fire_gather(nxt)

                drain_gather(s)
                mult_buf(s)
                fire_scatter(s)

                @pl.when(t >= 1)
                def _():
                    drain_scatter(prv)

                @pl.when(t < NCHUNKS - 2)
                def _():
                    fire_edges(prv, t + 2)
            return 0

        lax.fori_loop(0, NTRIP, triple, 0)
        drain_scatter((NCHUNKS - 1) % NBUF)

    pltpu.sync_copy(zeros_hbm,
                    acc_sh.at[pl.ds(sid * ROWS_PER_TILE, ROWS_PER_TILE)])
    plsc.subcore_barrier()

    @pl.when(core == 0)
    def _():
        run_pass(in0_hbm)

    @pl.when(core == 1)
    def _():
        run_pass(in1_hbm)

    plsc.subcore_barrier()

    pltpu.sync_copy(
        acc_sh.at[pl.ds(sid * ROWS_PER_TILE, ROWS_PER_TILE)],
        out_hbm.at[core, pl.ds(sid * ROWS_PER_TILE, ROWS_PER_TILE)])


_hop = functools.partial(
    pl.kernel,
    out_type=jax.ShapeDtypeStruct((2, NPAD, FH), jnp.float32),
    mesh=plsc.VectorSubcoreMesh(core_axis_name="c", subcore_axis_name="s"),
    compiler_params=pltpu.CompilerParams(use_tc_tiling_on_sc=False),
    scratch_types=[
        pltpu.VMEM_SHARED((NPAD, FH), jnp.float32),
        pltpu.VMEM((NBUF, CROWS, 3, EC), jnp.int32),
        pltpu.VMEM((NBUF, CE, FH), jnp.float32),
    ] + [pltpu.SemaphoreType.DMA] * (3 * NBUF),
)(_hop_body)


def kernel(x, adj_values, W1, b1, W2, b2, adj_indices):
    dst = adj_indices[0]
    src = adj_indices[1]

    epad = EPAD - E
    val_fix = (adj_values * jnp.float32(VSCALE)).astype(jnp.int32)
    srcp = jnp.concatenate([src, jnp.zeros((epad,), jnp.int32)]).reshape(EROWS, EC)
    dstp = jnp.concatenate([dst, jnp.zeros((epad,), jnp.int32)]).reshape(EROWS, EC)
    valp = jnp.concatenate([val_fix, jnp.zeros((epad,), jnp.int32)]).reshape(EROWS, EC)
    edges = jnp.stack([srcp, dstp, valp], axis=1)

    xp = jnp.pad(x, ((0, NPAD - N), (0, 0)))
    zeros_blob = jnp.zeros((ROWS_PER_TILE, FH), jnp.float32)

    h = _mlp(xp, W1, b1, W2, b2)
    for _ in range(K):
        h = _hop(edges, h[0], h[1], zeros_blob)
    return jnp.concatenate([h[0, :N], h[1, :N]], axis=1)
